# baseline (speedup 1.0000x reference)
import functools

import jax
import jax.numpy as jnp
from jax import lax
from jax.experimental import pallas as pl
from jax.experimental.pallas import tpu as pltpu
from jax.experimental.pallas import tpu_sc as plsc

_T = 100000
_D = 128
_L = 16
_MAX_NORM = 1.0


def _rsqrt16(x):
    i = plsc.bitcast(x, jnp.int32)
    y = plsc.bitcast(jnp.int32(0x5F3759DF) - (i >> 1), jnp.float32)
    for _ in range(3):
        y = y * (1.5 - 0.5 * x * y * y)
    return y


def _splat_sum(v):
    lane = lax.iota(jnp.int32, _L)
    for k in (8, 4, 2, 1):
        v = v + v.at[lane ^ k].get(mode="promise_in_bounds")
    return v


def _body(t_hbm, table_hbm, out_hbm, t_v, idx_v, rows_v, out_v, sem):
    lane = lax.iota(jnp.int32, _L)
    pltpu.async_copy(t_hbm.at[lane * 0], t_v, sem).wait()
    t16 = t_v[...]

    time = (t16 + 1.0) / 2.0 * float(_T - 1)
    t0 = time.astype(jnp.int32)
    alpha = time - t0.astype(jnp.float32)
    idx = jnp.minimum(t0 + jnp.minimum(lane, 1), _T - 1)
    idx_v[...] = idx
    pltpu.async_copy(table_hbm.at[idx_v.at[pl.ds(0, 2)]], rows_v, sem).wait()

    chunks0, chunks1 = [], []
    acc0 = jnp.zeros((_L,), jnp.float32)
    acc1 = jnp.zeros((_L,), jnp.float32)
    for c in range(_D // _L):
        ch0 = rows_v[0, pl.ds(c * _L, _L)]
        ch1 = rows_v[1, pl.ds(c * _L, _L)]
        chunks0.append(ch0)
        chunks1.append(ch1)
        acc0 = acc0 + ch0 * ch0
        acc1 = acc1 + ch1 * ch1
    scale0 = jnp.minimum(_MAX_NORM, _rsqrt16(_splat_sum(acc0)))
    scale1 = jnp.minimum(_MAX_NORM, _rsqrt16(_splat_sum(acc1)))

    for c in range(_D // _L):
        e0 = chunks0[c] * scale0
        e1 = chunks1[c] * scale1
        out_v[0, pl.ds(c * _L, _L)] = e0 + alpha * (e1 - e0)
    pltpu.sync_copy(out_v, out_hbm)


_sc_lerp = functools.partial(
    pl.kernel,
    out_type=jax.ShapeDtypeStruct((1, _D), jnp.float32),
    mesh=plsc.VectorSubcoreMesh(
        core_axis_name="c", subcore_axis_name="s", num_cores=1, num_subcores=1
    ),
    scratch_types=[
        pltpu.VMEM((_L,), jnp.float32),
        pltpu.VMEM((_L,), jnp.int32),
        pltpu.VMEM((2, _D), jnp.float32),
        pltpu.VMEM((1, _D), jnp.float32),
        pltpu.SemaphoreType.DMA,
    ],
    compiler_params=pltpu.CompilerParams(
        needs_layout_passes=False, skip_device_barrier=True
    ),
)(_body)


def kernel(t, time_emb_weight):
    return _sc_lerp(t, time_emb_weight)
